# baseline (speedup 1.0000x reference)
import functools

import jax
import jax.numpy as jnp
from jax import lax
from jax.experimental import pallas as pl
from jax.experimental.pallas import tpu as pltpu
from jax.experimental.pallas import tpu_sc as plsc

N = 10000
D = 256
H = 128
E = 160000
NT = 16
NC = 2
EB = 128
CH = 16
NCH = 5
NB = NCH * CH
E_PAD = NT * NB * EB
ZR = 632
N_ACC = NT * ZR
RPT = 624
TAIL = N - NT * RPT
NBLK = -(-N // 128)

_mesh = plsc.VectorSubcoreMesh(core_axis_name="c", subcore_axis_name="s")


NB0 = (NB + 1) // 2
NB1 = NB - NB0


@functools.partial(
    pl.kernel,
    out_type=jax.ShapeDtypeStruct((NC * N, H), jnp.float32),
    mesh=_mesh,
    scratch_types=[
        pltpu.VMEM((NB, EB), jnp.int32),
        pltpu.VMEM((EB, H), jnp.float32),
        pltpu.VMEM_SHARED((N_ACC, H), jnp.float32),
    ],
)
def _deg_sc(dst_hbm, ones_hbm, zeros_hbm, deg_out, dst_v, ones_v, dacc):
    c = lax.axis_index("c")
    s = lax.axis_index("s")
    pltpu.sync_copy(zeros_hbm, dacc.at[pl.ds(s * ZR, ZR)])
    pltpu.sync_copy(dst_hbm.at[s], dst_v)
    pltpu.sync_copy(ones_hbm, ones_v)
    plsc.subcore_barrier()

    def body(b, carry):
        pltpu.sync_copy(ones_v, dacc.at[dst_v.at[b]], add=True)
        return carry
    lax.fori_loop(c * NB0, NB0 + c * NB1, body, 0)

    plsc.subcore_barrier()
    pltpu.sync_copy(dacc.at[pl.ds(s * RPT, RPT)],
                    deg_out.at[pl.ds(c * N + s * RPT, RPT)])

    @pl.when(s == 0)
    def _tail():
        pltpu.sync_copy(dacc.at[pl.ds(NT * RPT, TAIL)],
                        deg_out.at[pl.ds(c * N + NT * RPT, TAIL)])


@functools.partial(
    pl.kernel,
    out_type=jax.ShapeDtypeStruct((NC * N, H), jnp.float32),
    mesh=_mesh,
    scratch_types=[
        pltpu.VMEM((NB, EB), jnp.int32),
        pltpu.VMEM((NB, EB), jnp.int32),
        pltpu.VMEM((EB, H), jnp.float32),
        pltpu.VMEM_SHARED((N_ACC, H), jnp.float32),
        pltpu.SemaphoreType.DMA,
    ],
)
def _agg_sc(g_hbm, src_hbm, dst_hbm, zeros_hbm, s_out,
            src_v, dst_v, rows_v, acc, sem):
    c = lax.axis_index("c")
    s = lax.axis_index("s")
    w = c * NT + s
    pltpu.sync_copy(zeros_hbm, acc.at[pl.ds(s * ZR, ZR)])
    pltpu.sync_copy(src_hbm.at[w], src_v)
    pltpu.sync_copy(dst_hbm.at[s], dst_v)
    plsc.subcore_barrier()

    def body(b, carry):
        pltpu.async_copy(g_hbm.at[src_v.at[b]], rows_v, sem).wait()
        pltpu.sync_copy(rows_v, acc.at[dst_v.at[b]], add=True)
        return carry
    lax.fori_loop(0, NB, body, 0)

    plsc.subcore_barrier()
    pltpu.sync_copy(acc.at[pl.ds(s * RPT, RPT)],
                    s_out.at[pl.ds(c * N + s * RPT, RPT)])

    @pl.when(s == 0)
    def _tail():
        pltpu.sync_copy(acc.at[pl.ds(NT * RPT, TAIL)],
                        s_out.at[pl.ds(c * N + NT * RPT, TAIL)])


def _dinv_of(deg_ref):
    return lax.rsqrt(deg_ref[0][:, 0:1] + deg_ref[1][:, 0:1] + 1.0)


def _mm_body(x_ref, w_ref, deg_ref, o_ref):
    h = jnp.dot(x_ref[...], w_ref[...], preferred_element_type=jnp.float32)
    o_ref[...] = (h * _dinv_of(deg_ref))[None]


def _mm_scaled(x, w, deg2):
    return pl.pallas_call(
        _mm_body,
        grid=(NBLK, NC),
        in_specs=[
            pl.BlockSpec((128, D), lambda i, c: (i, 0)),
            pl.BlockSpec((D, H), lambda i, c: (0, c)),
            pl.BlockSpec((NC, 128, H), lambda i, c: (0, i, 0)),
        ],
        out_specs=pl.BlockSpec((1, 128, H), lambda i, c: (c, i, 0)),
        out_shape=jax.ShapeDtypeStruct((NC, N, H), jnp.float32),
    )(x, w, deg2)


def _mid_body(s_ref, g_ref, deg_ref, b_ref, w_ref, o_ref):
    dinv = _dinv_of(deg_ref)
    t = jnp.concatenate([s_ref[0] + g_ref[0], s_ref[1] + g_ref[1]], axis=1)
    h = jnp.maximum(dinv * t + b_ref[...], 0.0)
    o = jnp.dot(h, w_ref[...], preferred_element_type=jnp.float32)
    o_ref[...] = (o * dinv)[None]


def _mid(s1, g1, deg2, b, w):
    return pl.pallas_call(
        _mid_body,
        grid=(NBLK, NC),
        in_specs=[
            pl.BlockSpec((NC, 128, H), lambda i, c: (0, i, 0)),
            pl.BlockSpec((NC, 128, H), lambda i, c: (0, i, 0)),
            pl.BlockSpec((NC, 128, H), lambda i, c: (0, i, 0)),
            pl.BlockSpec((1, D), lambda i, c: (0, 0)),
            pl.BlockSpec((D, H), lambda i, c: (0, c)),
        ],
        out_specs=pl.BlockSpec((1, 128, H), lambda i, c: (c, i, 0)),
        out_shape=jax.ShapeDtypeStruct((NC, N, H), jnp.float32),
    )(s1, g1, deg2, b, w)


def _out_body(s_ref, g_ref, deg_ref, b_ref, o_ref):
    dinv = _dinv_of(deg_ref)
    t = jnp.concatenate([s_ref[0] + g_ref[0], s_ref[1] + g_ref[1]], axis=1)
    o_ref[...] = dinv * t + b_ref[...]


def _final(s2, g2, deg2, b):
    return pl.pallas_call(
        _out_body,
        grid=(NBLK,),
        in_specs=[
            pl.BlockSpec((NC, 128, H), lambda i: (0, i, 0)),
            pl.BlockSpec((NC, 128, H), lambda i: (0, i, 0)),
            pl.BlockSpec((NC, 128, H), lambda i: (0, i, 0)),
            pl.BlockSpec((1, D), lambda i: (0, 0)),
        ],
        out_specs=pl.BlockSpec((128, D), lambda i: (i, 0)),
        out_shape=jax.ShapeDtypeStruct((N, D), jnp.float32),
    )(s2, g2, deg2, b)


def kernel(x, edge_index, W1, b1, W2, b2):
    src = edge_index[0].astype(jnp.int32)
    dst = edge_index[1].astype(jnp.int32)
    pad = E_PAD - E
    src_p = jnp.concatenate([src, jnp.zeros((pad,), jnp.int32)])
    dst_p = jnp.concatenate([dst, jnp.full((pad,), N, jnp.int32)])
    dst_t = dst_p.reshape(NT, NB, EB)
    src_t = src_p.reshape(NT, NB, EB)
    src_all = jnp.stack([src_t, src_t + N]).reshape(NC * NT, NB, EB)
    zeros_h = jnp.zeros((ZR, H), jnp.float32)
    ones_h = jnp.ones((EB, H), jnp.float32)
    b1r = b1.reshape(1, D)
    b2r = b2.reshape(1, D)

    deg2 = _deg_sc(dst_t, ones_h, zeros_h).reshape(NC, N, H)
    g1 = _mm_scaled(x, W1, deg2)
    s1 = _agg_sc(g1.reshape(NC * N, H), src_all, dst_t, zeros_h)
    g2 = _mid(s1.reshape(NC, N, H), g1, deg2, b1r, W2)
    s2 = _agg_sc(g2.reshape(NC * N, H), src_all, dst_t, zeros_h)
    return _final(s2.reshape(NC, N, H), g2, deg2, b2r)


# speedup vs baseline: 1.3332x; 1.3332x over previous
import functools

import jax
import jax.numpy as jnp
from jax import lax
from jax.experimental import pallas as pl
from jax.experimental.pallas import tpu as pltpu
from jax.experimental.pallas import tpu_sc as plsc

N = 10000
D = 256
H = 128
E = 160000
NT = 16
NC = 2
EB = 128
EPT = E // NT
NB = -(-EPT // EB)
PADT = NB * EB - EPT
ZR = 632
N_ACC = NT * ZR
RPT = 624
TAIL = N - NT * RPT
NBLK = -(-N // 128)

_mesh = plsc.VectorSubcoreMesh(core_axis_name="c", subcore_axis_name="s")


NB0 = (NB + 1) // 2
NB1 = NB - NB0


@functools.partial(
    pl.kernel,
    out_type=jax.ShapeDtypeStruct((NC * N, H), jnp.float32),
    mesh=_mesh,
    scratch_types=[
        pltpu.VMEM((NB, EB), jnp.int32),
        pltpu.VMEM((EB, H), jnp.float32),
        pltpu.VMEM_SHARED((N_ACC, H), jnp.float32),
    ],
)
def _deg_sc(dst_hbm, ones_hbm, zeros_hbm, deg_out, dst_v, ones_v, dacc):
    c = lax.axis_index("c")
    s = lax.axis_index("s")
    pltpu.sync_copy(zeros_hbm, dacc.at[pl.ds(s * ZR, ZR)])
    pltpu.sync_copy(dst_hbm.at[s], dst_v)
    pltpu.sync_copy(ones_hbm, ones_v)
    plsc.subcore_barrier()

    def body(b, carry):
        pltpu.sync_copy(ones_v, dacc.at[dst_v.at[b]], add=True)
        return carry
    lax.fori_loop(c * NB0, NB0 + c * NB1, body, 0)

    plsc.subcore_barrier()
    pltpu.sync_copy(dacc.at[pl.ds(s * RPT, RPT)],
                    deg_out.at[pl.ds(c * N + s * RPT, RPT)])

    @pl.when(s == 0)
    def _tail():
        pltpu.sync_copy(dacc.at[pl.ds(NT * RPT, TAIL)],
                        deg_out.at[pl.ds(c * N + NT * RPT, TAIL)])


@functools.partial(
    pl.kernel,
    out_type=jax.ShapeDtypeStruct((NC * N, H), jnp.float32),
    mesh=_mesh,
    scratch_types=[
        pltpu.VMEM((NB, EB), jnp.int32),
        pltpu.VMEM((NB, EB), jnp.int32),
        pltpu.VMEM((EB, H), jnp.float32),
        pltpu.VMEM_SHARED((N_ACC, H), jnp.float32),
        pltpu.SemaphoreType.DMA,
    ],
)
def _agg_sc(g_hbm, src_hbm, dst_hbm, zeros_hbm, s_out,
            src_v, dst_v, rows_v, acc, sem):
    c = lax.axis_index("c")
    s = lax.axis_index("s")
    w = c * NT + s
    pltpu.sync_copy(zeros_hbm, acc.at[pl.ds(s * ZR, ZR)])
    pltpu.sync_copy(src_hbm.at[w], src_v)
    pltpu.sync_copy(dst_hbm.at[s], dst_v)
    plsc.subcore_barrier()

    def body(b, carry):
        pltpu.async_copy(g_hbm.at[src_v.at[b]], rows_v, sem).wait()
        pltpu.sync_copy(rows_v, acc.at[dst_v.at[b]], add=True)
        return carry
    lax.fori_loop(0, NB, body, 0)

    plsc.subcore_barrier()
    pltpu.sync_copy(acc.at[pl.ds(s * RPT, RPT)],
                    s_out.at[pl.ds(c * N + s * RPT, RPT)])

    @pl.when(s == 0)
    def _tail():
        pltpu.sync_copy(acc.at[pl.ds(NT * RPT, TAIL)],
                        s_out.at[pl.ds(c * N + NT * RPT, TAIL)])


def _dinv_of(deg_ref):
    return lax.rsqrt(deg_ref[0][:, 0:1] + deg_ref[1][:, 0:1] + 1.0)


def _mm_body(x_ref, w_ref, deg_ref, o_ref):
    h = jnp.dot(x_ref[...], w_ref[...], preferred_element_type=jnp.float32)
    o_ref[...] = (h * _dinv_of(deg_ref))[None]


def _mm_scaled(x, w, deg2):
    return pl.pallas_call(
        _mm_body,
        grid=(NBLK, NC),
        in_specs=[
            pl.BlockSpec((128, D), lambda i, c: (i, 0)),
            pl.BlockSpec((D, H), lambda i, c: (0, c)),
            pl.BlockSpec((NC, 128, H), lambda i, c: (0, i, 0)),
        ],
        out_specs=pl.BlockSpec((1, 128, H), lambda i, c: (c, i, 0)),
        out_shape=jax.ShapeDtypeStruct((NC, N, H), jnp.float32),
    )(x, w, deg2)


def _mid_body(s_ref, g_ref, deg_ref, b_ref, w_ref, o_ref):
    dinv = _dinv_of(deg_ref)
    t = jnp.concatenate([s_ref[0] + g_ref[0], s_ref[1] + g_ref[1]], axis=1)
    h = jnp.maximum(dinv * t + b_ref[...], 0.0)
    o = jnp.dot(h, w_ref[...], preferred_element_type=jnp.float32)
    o_ref[...] = (o * dinv)[None]


def _mid(s1, g1, deg2, b, w):
    return pl.pallas_call(
        _mid_body,
        grid=(NBLK, NC),
        in_specs=[
            pl.BlockSpec((NC, 128, H), lambda i, c: (0, i, 0)),
            pl.BlockSpec((NC, 128, H), lambda i, c: (0, i, 0)),
            pl.BlockSpec((NC, 128, H), lambda i, c: (0, i, 0)),
            pl.BlockSpec((1, D), lambda i, c: (0, 0)),
            pl.BlockSpec((D, H), lambda i, c: (0, c)),
        ],
        out_specs=pl.BlockSpec((1, 128, H), lambda i, c: (c, i, 0)),
        out_shape=jax.ShapeDtypeStruct((NC, N, H), jnp.float32),
    )(s1, g1, deg2, b, w)


def _out_body(s_ref, g_ref, deg_ref, b_ref, o_ref):
    dinv = _dinv_of(deg_ref)
    t = jnp.concatenate([s_ref[0] + g_ref[0], s_ref[1] + g_ref[1]], axis=1)
    o_ref[...] = dinv * t + b_ref[...]


def _final(s2, g2, deg2, b):
    return pl.pallas_call(
        _out_body,
        grid=(NBLK,),
        in_specs=[
            pl.BlockSpec((NC, 128, H), lambda i: (0, i, 0)),
            pl.BlockSpec((NC, 128, H), lambda i: (0, i, 0)),
            pl.BlockSpec((NC, 128, H), lambda i: (0, i, 0)),
            pl.BlockSpec((1, D), lambda i: (0, 0)),
        ],
        out_specs=pl.BlockSpec((128, D), lambda i: (i, 0)),
        out_shape=jax.ShapeDtypeStruct((N, D), jnp.float32),
    )(s2, g2, deg2, b)


def kernel(x, edge_index, W1, b1, W2, b2):
    src = edge_index[0].astype(jnp.int32)
    dst = edge_index[1].astype(jnp.int32)
    src_t = jnp.pad(src.reshape(NT, EPT),
                    ((0, 0), (0, PADT))).reshape(NT, NB, EB)
    dump = jnp.broadcast_to(N + jnp.arange(PADT, dtype=jnp.int32),
                            (NT, PADT))
    dst_t = jnp.concatenate([dst.reshape(NT, EPT), dump],
                            axis=1).reshape(NT, NB, EB)
    src_all = jnp.stack([src_t, src_t + N]).reshape(NC * NT, NB, EB)
    zeros_h = jnp.zeros((ZR, H), jnp.float32)
    ones_h = jnp.ones((EB, H), jnp.float32)
    b1r = b1.reshape(1, D)
    b2r = b2.reshape(1, D)

    deg2 = _deg_sc(dst_t, ones_h, zeros_h).reshape(NC, N, H)
    g1 = _mm_scaled(x, W1, deg2)
    s1 = _agg_sc(g1.reshape(NC * N, H), src_all, dst_t, zeros_h)
    g2 = _mid(s1.reshape(NC, N, H), g1, deg2, b1r, W2)
    s2 = _agg_sc(g2.reshape(NC * N, H), src_all, dst_t, zeros_h)
    return _final(s2.reshape(NC, N, H), g2, deg2, b2r)
